# baseline (speedup 1.0000x reference)
import functools

import jax
import jax.numpy as jnp
from jax import lax
from jax.experimental import pallas as pl
from jax.experimental.pallas import tpu as pltpu
from jax.experimental.pallas import tpu_sc as plsc

_ALPHA = 0.001
_BETA = 0.001
_S = 3
_R = 16
_DPF = _R * (1 + 2 * _S)
_BATCH = 16384
_NC, _NS, _L = 2, 16, 16
_NW = _NC * _NS
_BPW = _BATCH // _NW
_NG = _BPW // _L
_ICH = 128
_CHUNK = 64
_NCH = _BPW // _CHUNK
_NGC = _CHUNK // _L


def _mf_body(ijk_hbm, pF_hbm, M8_hbm, out_hbm,
             ijk_buf,
             idx_i, idx_j, idx_k0, idx_k1, idx_k2,
             gj, gk0, gk1, gk2,
             pf_buf, mj_buf, mk0_buf, mk1_buf, mk2_buf, out_buf,
             s0, s1, s2, s3, s4):
    wid = lax.axis_index("s") * _NC + lax.axis_index("c")
    base = wid * _BPW

    lane = lax.iota(jnp.int32, _L)

    def col(c):
        return jnp.full((_L,), c, dtype=jnp.int32)

    def stage(tch, carry):
        pltpu.sync_copy(ijk_hbm.at[pl.ds(base + tch * _ICH, _ICH)], ijk_buf)
        for grp in range(_ICH // _L):
            rows = grp * _L + lane
            doff = pl.multiple_of(tch * _ICH + grp * _L, _L)
            for c, buf in enumerate((idx_i, idx_j, idx_k0, idx_k1, idx_k2)):
                buf[pl.ds(doff, _L)] = plsc.load_gather(
                    ijk_buf, [rows, col(c)])
        return carry

    lax.fori_loop(0, _BPW // _ICH, stage, 0)

    def pf_fetch(g, carry):
        goff = pl.multiple_of(g * _L, _L)
        ivec = idx_i[pl.ds(goff, _L)]
        for l in range(_L):
            n = g * _L + l
            pltpu.make_async_copy(
                pF_hbm.at[pl.ds(ivec[l], 1)], pf_buf.at[pl.ds(n, 1)],
                s0).start()
        return carry

    lax.fori_loop(0, _NG, pf_fetch, 0)
    pltpu.make_async_copy(pF_hbm.at[pl.ds(0, _BPW)], pf_buf, s0).wait()

    def chunk_body(ch, carry):
        cb = ch * _CHUNK
        for idx_src, gdst in ((idx_j, gj), (idx_k0, gk0),
                              (idx_k1, gk1), (idx_k2, gk2)):
            for v in range(_CHUNK // _L):
                off = pl.multiple_of(cb + v * _L, _L)
                gdst[pl.ds(v * _L, _L)] = (
                    idx_src[pl.ds(off, _L)] >> 3)
        cps = [
            pltpu.async_copy(M8_hbm.at[gj], mj_buf, s1),
            pltpu.async_copy(M8_hbm.at[gk0], mk0_buf, s2),
            pltpu.async_copy(M8_hbm.at[gk1], mk1_buf, s3),
            pltpu.async_copy(M8_hbm.at[gk2], mk2_buf, s4),
        ]
        for cp in cps:
            cp.wait()

        def group(grp, carry2):
            slot = grp * _L + lane
            grow = cb + slot
            goff = pl.multiple_of(cb + grp * _L, _L)
            jv = idx_j[pl.ds(goff, _L)]
            k0v = idx_k0[pl.ds(goff, _L)]
            k1v = idx_k1[pl.ds(goff, _L)]
            k2v = idx_k2[pl.ds(goff, _L)]
            joff = (jv & 7) << 4
            k0off = (k0v & 7) << 4
            k1off = (k1v & 7) << 4
            k2off = (k2v & 7) << 4

            mj = [plsc.load_gather(mj_buf, [slot, joff + r])
                  for r in range(_R)]
            ms = [
                plsc.load_gather(mk0_buf, [slot, k0off + r])
                + plsc.load_gather(mk1_buf, [slot, k1off + r])
                + plsc.load_gather(mk2_buf, [slot, k2off + r])
                for r in range(_R)
            ]

            accp = None
            a = [None, None, None]
            b = [None, None, None]
            for c in range(_DPF):
                pv = plsc.load_gather(pf_buf, [grow, col(c)])
                if c < _R:
                    term = pv * mj[c]
                    accp = term if accp is None else accp + term
                elif c < (1 + _S) * _R:
                    r, s = divmod(c - _R, _S)
                    term = pv * mj[r]
                    a[s] = term if a[s] is None else a[s] + term
                else:
                    r, s = divmod(c - (1 + _S) * _R, _S)
                    term = pv * ms[r]
                    b[s] = term if b[s] is None else b[s] + term

            res = _ALPHA * accp + (_BETA * _BETA) * (
                a[0] * b[0] + a[1] * b[1] + a[2] * b[2])
            plsc.store_scatter(out_buf, [grow], res)
            return carry2

        lax.fori_loop(0, _NGC, group, 0)
        return carry

    lax.fori_loop(0, _NCH, chunk_body, 0)

    pltpu.sync_copy(out_buf, out_hbm.at[pl.ds(base, _BPW)])


@jax.jit
def _mf_call(ijk, pF, M8):
    mesh = plsc.VectorSubcoreMesh(core_axis_name="c", subcore_axis_name="s")
    f = functools.partial(
        pl.kernel,
        mesh=mesh,
        out_type=jax.ShapeDtypeStruct((_BATCH,), jnp.float32),
        compiler_params=pltpu.CompilerParams(
            use_tc_tiling_on_sc=True, needs_layout_passes=False),
        scratch_types=[
            pltpu.VMEM((_ICH, 5), jnp.int32),
            pltpu.VMEM((_BPW,), jnp.int32),
            pltpu.VMEM((_BPW,), jnp.int32),
            pltpu.VMEM((_BPW,), jnp.int32),
            pltpu.VMEM((_BPW,), jnp.int32),
            pltpu.VMEM((_BPW,), jnp.int32),
            pltpu.VMEM((_CHUNK,), jnp.int32),
            pltpu.VMEM((_CHUNK,), jnp.int32),
            pltpu.VMEM((_CHUNK,), jnp.int32),
            pltpu.VMEM((_CHUNK,), jnp.int32),
            pltpu.VMEM((_BPW, _DPF), jnp.float32),
            pltpu.VMEM((_CHUNK, 128), jnp.float32),
            pltpu.VMEM((_CHUNK, 128), jnp.float32),
            pltpu.VMEM((_CHUNK, 128), jnp.float32),
            pltpu.VMEM((_CHUNK, 128), jnp.float32),
            pltpu.VMEM((_BPW,), jnp.float32),
            pltpu.SemaphoreType.DMA,
            pltpu.SemaphoreType.DMA,
            pltpu.SemaphoreType.DMA,
            pltpu.SemaphoreType.DMA,
            pltpu.SemaphoreType.DMA,
        ],
    )(_mf_body)
    return f(ijk, pF, M8)


def kernel(ijk, pF, M):
    return _mf_call(ijk, pF, M.reshape(-1, 128))


# speedup vs baseline: 1.1157x; 1.1157x over previous
import functools

import jax
import jax.numpy as jnp
from jax import lax
from jax.experimental import pallas as pl
from jax.experimental.pallas import tpu as pltpu
from jax.experimental.pallas import tpu_sc as plsc

_ALPHA = 0.001
_BETA = 0.001
_S = 3
_R = 16
_DPF = _R * (1 + 2 * _S)
_BATCH = 16384
_NC, _NS, _L = 2, 16, 16
_NW = _NC * _NS
_BPW = _BATCH // _NW
_NG = _BPW // _L
_CHUNK = 64
_NCH = _BPW // _CHUNK
_NGC = _CHUNK // _L
_MMW = 2 * _R

_lane = None


def _col(c):
    return jnp.full((_L,), c, dtype=jnp.int32)


def _k1_body(ijkT_hbm, M_hbm, mm_hbm,
             ijkT_buf, mjb, k0b, k1b, k2b, stage, sm):
    wid = lax.axis_index("s") * _NC + lax.axis_index("c")
    base = wid * _BPW
    lane = lax.iota(jnp.int32, _L)

    pltpu.sync_copy(ijkT_hbm.at[:, pl.ds(base, _BPW)], ijkT_buf)

    def chunk_body(ch, carry):
        cb = ch * _CHUNK
        for grp in range(_NGC):
            slotv = cb + grp * _L + lane
            for q, buf in ((1, mjb), (2, k0b), (3, k1b), (4, k2b)):
                vec = plsc.load_gather(ijkT_buf, [_col(q), slotv])
                for l in range(_L):
                    pltpu.make_async_copy(
                        M_hbm.at[pl.ds(vec[l], 1)],
                        buf.at[pl.ds(grp * _L + l, 1)], sm).start()
        for buf in (mjb, k0b, k1b, k2b):
            pltpu.make_async_copy(M_hbm.at[pl.ds(0, _CHUNK)], buf, sm).wait()

        for grp in range(_NGC):
            slot = grp * _L + lane
            flat = (cb + slot) * _MMW
            for r in range(_R):
                mj = plsc.load_gather(mjb, [slot, _col(r)])
                ms = (plsc.load_gather(k0b, [slot, _col(r)])
                      + plsc.load_gather(k1b, [slot, _col(r)])
                      + plsc.load_gather(k2b, [slot, _col(r)]))
                plsc.store_scatter(stage, [flat + r], mj)
                plsc.store_scatter(stage, [flat + (_R + r)], ms)
        return carry

    lax.fori_loop(0, _NCH, chunk_body, 0)

    pltpu.sync_copy(stage, mm_hbm.at[pl.ds(base * _MMW, _BPW * _MMW)])


def _k2_body(ijkT_hbm, pF_hbm, mm_hbm, out_hbm,
             ijkT_buf, pf_buf, mm_buf, out_buf, sp):
    wid = lax.axis_index("s") * _NC + lax.axis_index("c")
    base = wid * _BPW
    lane = lax.iota(jnp.int32, _L)

    pltpu.sync_copy(ijkT_hbm.at[:, pl.ds(base, _BPW)], ijkT_buf)

    def pf_fetch(g, carry):
        ivec = plsc.load_gather(ijkT_buf, [_col(0), g * _L + lane])
        for l in range(_L):
            pltpu.make_async_copy(
                pF_hbm.at[pl.ds(ivec[l], 1)],
                pf_buf.at[pl.ds(g * _L + l, 1)], sp).start()
        return carry

    lax.fori_loop(0, _NG, pf_fetch, 0)

    pltpu.sync_copy(mm_hbm.at[pl.ds(base * _MMW, _BPW * _MMW)], mm_buf)
    pltpu.make_async_copy(pF_hbm.at[pl.ds(0, _BPW)], pf_buf, sp).wait()

    def group(grp, carry):
        grow = grp * _L + lane
        mbase = grow * _MMW

        mj = [plsc.load_gather(mm_buf, [mbase + r]) for r in range(_R)]
        ms = [plsc.load_gather(mm_buf, [mbase + (_R + r)])
              for r in range(_R)]

        accp = None
        a = [None, None, None]
        b = [None, None, None]
        for c in range(_DPF):
            pv = plsc.load_gather(pf_buf, [grow, _col(c)])
            if c < _R:
                term = pv * mj[c]
                accp = term if accp is None else accp + term
            elif c < (1 + _S) * _R:
                r, s = divmod(c - _R, _S)
                term = pv * mj[r]
                a[s] = term if a[s] is None else a[s] + term
            else:
                r, s = divmod(c - (1 + _S) * _R, _S)
                term = pv * ms[r]
                b[s] = term if b[s] is None else b[s] + term

        res = _ALPHA * accp + (_BETA * _BETA) * (
            a[0] * b[0] + a[1] * b[1] + a[2] * b[2])
        plsc.store_scatter(out_buf, [grow], res)
        return carry

    lax.fori_loop(0, _NG, group, 0)

    pltpu.sync_copy(out_buf, out_hbm.at[pl.ds(base, _BPW)])


@jax.jit
def _mf_call(ijkT, pF, M):
    mesh = plsc.VectorSubcoreMesh(core_axis_name="c", subcore_axis_name="s")
    params = pltpu.CompilerParams(
        use_tc_tiling_on_sc=True, needs_layout_passes=False)

    k1 = functools.partial(
        pl.kernel,
        mesh=mesh,
        out_type=jax.ShapeDtypeStruct((_BATCH * _MMW,), jnp.float32),
        compiler_params=params,
        scratch_types=[
            pltpu.VMEM((5, _BPW), jnp.int32),
            pltpu.VMEM((_CHUNK, _R), jnp.float32),
            pltpu.VMEM((_CHUNK, _R), jnp.float32),
            pltpu.VMEM((_CHUNK, _R), jnp.float32),
            pltpu.VMEM((_CHUNK, _R), jnp.float32),
            pltpu.VMEM((_BPW * _MMW,), jnp.float32),
            pltpu.SemaphoreType.DMA,
        ],
    )(_k1_body)
    mm = k1(ijkT, M)

    k2 = functools.partial(
        pl.kernel,
        mesh=mesh,
        out_type=jax.ShapeDtypeStruct((_BATCH,), jnp.float32),
        compiler_params=params,
        scratch_types=[
            pltpu.VMEM((5, _BPW), jnp.int32),
            pltpu.VMEM((_BPW, _DPF), jnp.float32),
            pltpu.VMEM((_BPW * _MMW,), jnp.float32),
            pltpu.VMEM((_BPW,), jnp.float32),
            pltpu.SemaphoreType.DMA,
        ],
    )(_k2_body)
    return k2(ijkT, pF, mm)


def kernel(ijk, pF, M):
    return _mf_call(ijk.T, pF, M)
